# baseline (speedup 1.0000x reference)
import functools

import numpy as np
import jax
import jax.numpy as jnp
from jax import lax
from jax.experimental import pallas as pl
from jax.experimental.pallas import tpu as pltpu

_POS_INDICES = [120, 124, 128, 132, 136, 140, 144, 148, 152, 237, 241, 245, 249, 253, 257, 261, 265, 269, 354, 358, 362, 366, 370, 374, 378, 382, 386, 471, 475, 479, 483, 487, 491, 495, 499, 503, 588, 592, 596, 600, 604, 608, 612, 616, 620, 705, 709, 713, 717, 721, 725, 729, 733, 737, 822, 826, 830, 834, 838, 842, 846, 850, 854, 48, 53, 60, 67, 73, 157, 352, 388, 583, 586, 817, 901, 906, 913, 920, 926]

_N = 79
_BB = 128


def _positions() -> np.ndarray:
    height, width = 25, 39
    pf = np.zeros((_N, 30), dtype=np.float32)
    mults = [2, 5, 12, 30, 100, 100, 100]
    for i, p in enumerate(_POS_INDICES):
        x = p % width
        y = p // width
        xn = x / (width - 1)
        yn = y / (height - 1)
        pf[i, 0] = xn
        pf[i, 1] = yn
        for k, m in enumerate(mults):
            c = 2 + 4 * k
            pf[i, c + 0] = np.sin(xn * m * np.pi) + 1
            pf[i, c + 1] = np.cos(xn * m * np.pi) + 1
            pf[i, c + 2] = np.sin(yn * m * np.pi) + 1
            pf[i, c + 3] = np.cos(yn * m * np.pi) + 1
    return pf


def _mm(a, b):
    return lax.dot_general(a, b, (((1,), (0,)), ((), ())),
                           preferred_element_type=jnp.float32)


def _cmm(x, w):
    n, b, ci = x.shape
    h = lax.dot_general(x.reshape(n * b, ci), w, (((1,), (0,)), ((), ())),
                        preferred_element_type=jnp.float32)
    return h.reshape(n, b, w.shape[1])


def _amm(A, h):
    return lax.dot_general(A, h, (((1,), (0,)), ((), ())),
                           preferred_element_type=jnp.float32)


def _body(state_ref, src_ref, dst_ref, pos_ref,
          w1a_ref, w1b_ref, w2_ref, w3_ref, b1_ref, b2_ref, b3_ref,
          l1g_ref, l1s_ref, l1p_ref, l1t_ref, lb1_ref,
          lw2_ref, lb2_ref, lw3_ref, lb3_ref,
          act_ref, reg_ref, A_s, *, nb):
    i = pl.program_id(0)

    @pl.when(i == 0)
    def _build_a():
        E = src_ref.shape[0]
        niota = lax.broadcasted_iota(jnp.int32, (E, _N), 1)
        srcb = (src_ref[...] == niota).astype(jnp.float32)
        dstb = (dst_ref[...] == niota).astype(jnp.float32)
        deg = jnp.sum(dstb, axis=0, keepdims=True) + 1.0
        dis = lax.rsqrt(deg)
        dsrc = lax.dot_general(srcb, dis, (((1,), (1,)), ((), ())),
                               preferred_element_type=jnp.float32)
        ddst = lax.dot_general(dstb, dis, (((1,), (1,)), ((), ())),
                               preferred_element_type=jnp.float32)
        nrm = dsrc * ddst
        Amat = lax.dot_general(dstb, srcb * nrm, (((0,), (0,)), ((), ())),
                               preferred_element_type=jnp.float32)
        ri = lax.broadcasted_iota(jnp.int32, (_N, _N), 0)
        ci = lax.broadcasted_iota(jnp.int32, (_N, _N), 1)
        ey = (ri == ci).astype(jnp.float32)
        A_s[...] = Amat + ey * (dis * dis)

    A = A_s[...]
    bf = jnp.bfloat16
    A_hi = A.astype(bf)
    A_lo = (A - A_hi.astype(jnp.float32)).astype(bf)
    x0f = state_ref[...]
    posm = pos_ref[...]

    posw1 = _mm(posm, w1b_ref[...])
    posl1 = _mm(posm, l1p_ref[...])

    def _aap(h):
        hbf = h.astype(bf)
        return _amm(A_hi, hbf) + _amm(A_lo, hbf)

    def _net(x0):
        h = _cmm(x0, w1a_ref[...]) + posw1[:, None, :]
        out = jax.nn.relu(_aap(h) + b1_ref[...][None])
        y = _cmm(out, l1g_ref[0:128, :])

        for l, (w_ref, b_ref) in enumerate(
                [(w2_ref, b2_ref), (w3_ref, b3_ref), (w3_ref, b3_ref), (w3_ref, b3_ref)]):
            h = _cmm(out, w_ref[...])
            out = jax.nn.relu(_aap(h) + b_ref[...][None])
            lo = 128 * (l + 1)
            y = y + _cmm(out, l1g_ref[lo:lo + 128, :])

        y = y + _cmm(x0, l1s_ref[...])
        y = y + posl1[:, None, :]
        ta = jnp.sum(x0[:, :, 1:2], axis=0)
        y = y + ta[None] * l1t_ref[...][None]
        y = y + lb1_ref[...][None]

        x2 = jnp.maximum(y, 0.01 * y)
        y2 = _cmm(x2, lw2_ref[...]) + lb2_ref[...][None]
        x3 = jnp.maximum(y2, 0.01 * y2)
        y3 = jnp.sum(x3 * lw3_ref[...][None], axis=2) + lb3_ref[...]
        return jnp.maximum(y3, 0.0) + jnp.log1p(jnp.exp(-jnp.abs(y3)))

    nch = 2
    hb = _BB // nch
    concs = [_net(x0f[:, k * hb:(k + 1) * hb, :]) for k in range(nch)]
    for k, ck in enumerate(concs):
        act_ref[k * hb:(k + 1) * hb, :] = (
            ck / (jnp.sum(ck, axis=0, keepdims=True) + 1e-20)).T

    @pl.when(i == 0)
    def _zero():
        reg_ref[...] = jnp.zeros_like(reg_ref)
    reg_ref[...] = reg_ref[...] + sum(jnp.sum(ck) for ck in concs)
    @pl.when(i == nb - 1)
    def _fin():
        reg_ref[...] = reg_ref[...] * (1.0 / (nb * _BB * _N))


def kernel(state, edge_index, W1, b1, W2, b2, W3, b3, W4, b4, W5, b5,
           lw1, lb1, lw2, lb2, lw3, lb3, deterministic=1):
    B = state.shape[0]
    E = edge_index.shape[1]
    nb = B // _BB

    state_t = jnp.transpose(state, (1, 0, 2))
    src = edge_index[0].reshape(E, 1)
    dst = edge_index[1].reshape(E, 1)
    pos = jnp.asarray(_positions())

    w1a, w1b = W1[:98, :], W1[98:, :]
    l1g = lw1[:640, :]
    l1s = lw1[640:738, :]
    l1p = lw1[738:768, :]
    l1t = lw1[768:769, :]

    full = lambda shp: pl.BlockSpec(shp, lambda i: tuple(0 for _ in shp))
    action, regsum = pl.pallas_call(
        functools.partial(_body, nb=nb),
        grid=(nb,),
        in_specs=[
            pl.BlockSpec((_N, _BB, 98), lambda i: (0, i, 0)),
            full((E, 1)), full((E, 1)), full((_N, 30)),
            full((98, 128)), full((30, 128)), full((128, 128)), full((128, 128)),
            full((1, 128)), full((1, 128)), full((1, 128)),
            full((640, 256)), full((98, 256)), full((30, 256)), full((1, 256)),
            full((1, 256)), full((256, 256)), full((1, 256)), full((1, 256)),
            full((1, 1)),
        ],
        out_specs=[
            pl.BlockSpec((_BB, _N), lambda i: (i, 0)),
            pl.BlockSpec((1, 1), lambda i: (0, 0)),
        ],
        out_shape=[
            jax.ShapeDtypeStruct((B, _N), jnp.float32),
            jax.ShapeDtypeStruct((1, 1), jnp.float32),
        ],
        scratch_shapes=[pltpu.VMEM((_N, _N), jnp.float32)],
    )(state_t, src, dst, pos,
      w1a, w1b, W2, W3,
      b1.reshape(1, 128), b2.reshape(1, 128), b3.reshape(1, 128),
      l1g, l1s, l1p, l1t, lb1.reshape(1, 256),
      lw2, lb2.reshape(1, 256), lw3.reshape(1, 256),
      lb3.reshape(1, 1))
    return action, regsum[0, 0]


# speedup vs baseline: 1.5948x; 1.5948x over previous
import functools

import numpy as np
import jax
import jax.numpy as jnp
from jax import lax
from jax.experimental import pallas as pl
from jax.experimental.pallas import tpu as pltpu

_POS_INDICES = [120, 124, 128, 132, 136, 140, 144, 148, 152, 237, 241, 245, 249, 253, 257, 261, 265, 269, 354, 358, 362, 366, 370, 374, 378, 382, 386, 471, 475, 479, 483, 487, 491, 495, 499, 503, 588, 592, 596, 600, 604, 608, 612, 616, 620, 705, 709, 713, 717, 721, 725, 729, 733, 737, 822, 826, 830, 834, 838, 842, 846, 850, 854, 48, 53, 60, 67, 73, 157, 352, 388, 583, 586, 817, 901, 906, 913, 920, 926]

_N = 79
_BB = 128


def _positions() -> np.ndarray:
    height, width = 25, 39
    pf = np.zeros((_N, 30), dtype=np.float32)
    mults = [2, 5, 12, 30, 100, 100, 100]
    for i, p in enumerate(_POS_INDICES):
        x = p % width
        y = p // width
        xn = x / (width - 1)
        yn = y / (height - 1)
        pf[i, 0] = xn
        pf[i, 1] = yn
        for k, m in enumerate(mults):
            c = 2 + 4 * k
            pf[i, c + 0] = np.sin(xn * m * np.pi) + 1
            pf[i, c + 1] = np.cos(xn * m * np.pi) + 1
            pf[i, c + 2] = np.sin(yn * m * np.pi) + 1
            pf[i, c + 3] = np.cos(yn * m * np.pi) + 1
    return pf


def _mm(a, b):
    return lax.dot_general(a, b, (((1,), (0,)), ((), ())),
                           preferred_element_type=jnp.float32)


def _cmm(x, w):
    n, b, ci = x.shape
    h = lax.dot_general(x.reshape(n * b, ci), w, (((1,), (0,)), ((), ())),
                        preferred_element_type=jnp.float32)
    return h.reshape(n, b, w.shape[1])


def _amm(A, h):
    return lax.dot_general(A, h, (((1,), (0,)), ((), ())),
                           preferred_element_type=jnp.float32)


def _body(state_ref, src_ref, dst_ref, pos_ref,
          w1a_ref, w1b_ref, w2_ref, w3_ref, b1_ref, b2_ref, b3_ref,
          l1g_ref, l1s_ref, l1p_ref, l1t_ref, lb1_ref,
          lw2_ref, lb2_ref, lw3_ref, lb3_ref,
          act_ref, reg_ref, A_s, *, nb):
    i = pl.program_id(0)

    @pl.when(i == 0)
    def _build_a():
        E = src_ref.shape[0]
        niota = lax.broadcasted_iota(jnp.int32, (E, _N), 1)
        srcb = (src_ref[...] == niota).astype(jnp.float32)
        dstb = (dst_ref[...] == niota).astype(jnp.float32)
        deg = jnp.sum(dstb, axis=0, keepdims=True) + 1.0
        dis = lax.rsqrt(deg)
        dsrc = lax.dot_general(srcb, dis, (((1,), (1,)), ((), ())),
                               preferred_element_type=jnp.float32)
        ddst = lax.dot_general(dstb, dis, (((1,), (1,)), ((), ())),
                               preferred_element_type=jnp.float32)
        nrm = dsrc * ddst
        Amat = lax.dot_general(dstb, srcb * nrm, (((0,), (0,)), ((), ())),
                               preferred_element_type=jnp.float32)
        ri = lax.broadcasted_iota(jnp.int32, (_N, _N), 0)
        ci = lax.broadcasted_iota(jnp.int32, (_N, _N), 1)
        ey = (ri == ci).astype(jnp.float32)
        A_s[...] = Amat + ey * (dis * dis)

    A = A_s[...]
    x0f = state_ref[...]
    posm = pos_ref[...]

    posw1 = _mm(posm, w1b_ref[...])
    posl1 = _mm(posm, l1p_ref[...])

    def _net(x0):
        h = _cmm(x0, w1a_ref[...]) + posw1[:, None, :]
        out = jax.nn.relu(_amm(A, h) + b1_ref[...][None])
        y = _cmm(out, l1g_ref[0:128, :])

        for l, (w_ref, b_ref) in enumerate(
                [(w2_ref, b2_ref), (w3_ref, b3_ref), (w3_ref, b3_ref), (w3_ref, b3_ref)]):
            h = _cmm(out, w_ref[...])
            out = jax.nn.relu(_amm(A, h) + b_ref[...][None])
            lo = 128 * (l + 1)
            y = y + _cmm(out, l1g_ref[lo:lo + 128, :])

        y = y + _cmm(x0, l1s_ref[...])
        y = y + posl1[:, None, :]
        ta = jnp.sum(x0[:, :, 1:2], axis=0)
        y = y + ta[None] * l1t_ref[...][None]
        y = y + lb1_ref[...][None]

        x2 = jnp.maximum(y, 0.01 * y)
        y2 = _cmm(x2, lw2_ref[...]) + lb2_ref[...][None]
        x3 = jnp.maximum(y2, 0.01 * y2)
        y3 = jnp.sum(x3 * lw3_ref[...][None], axis=2) + lb3_ref[...]
        return jnp.maximum(y3, 0.0) + jnp.log1p(jnp.exp(-jnp.abs(y3)))

    nch = 2
    hb = _BB // nch
    concs = [_net(x0f[:, k * hb:(k + 1) * hb, :]) for k in range(nch)]
    for k, ck in enumerate(concs):
        act_ref[:, k * hb:(k + 1) * hb] = (
            ck / (jnp.sum(ck, axis=0, keepdims=True) + 1e-20))

    @pl.when(i == 0)
    def _zero():
        reg_ref[...] = jnp.zeros_like(reg_ref)
    reg_ref[...] = reg_ref[...] + sum(jnp.sum(ck) for ck in concs)
    @pl.when(i == nb - 1)
    def _fin():
        reg_ref[...] = reg_ref[...] * (1.0 / (nb * _BB * _N))


def kernel(state, edge_index, W1, b1, W2, b2, W3, b3, W4, b4, W5, b5,
           lw1, lb1, lw2, lb2, lw3, lb3, deterministic=1):
    B = state.shape[0]
    E = edge_index.shape[1]
    nb = B // _BB

    state_t = jnp.transpose(state, (1, 0, 2))
    src = edge_index[0].reshape(E, 1)
    dst = edge_index[1].reshape(E, 1)
    pos = jnp.asarray(_positions())

    w1a, w1b = W1[:98, :], W1[98:, :]
    l1g = lw1[:640, :]
    l1s = lw1[640:738, :]
    l1p = lw1[738:768, :]
    l1t = lw1[768:769, :]

    full = lambda shp: pl.BlockSpec(shp, lambda i: tuple(0 for _ in shp))
    action, regsum = pl.pallas_call(
        functools.partial(_body, nb=nb),
        grid=(nb,),
        in_specs=[
            pl.BlockSpec((_N, _BB, 98), lambda i: (0, i, 0)),
            full((E, 1)), full((E, 1)), full((_N, 30)),
            full((98, 128)), full((30, 128)), full((128, 128)), full((128, 128)),
            full((1, 128)), full((1, 128)), full((1, 128)),
            full((640, 256)), full((98, 256)), full((30, 256)), full((1, 256)),
            full((1, 256)), full((256, 256)), full((1, 256)), full((1, 256)),
            full((1, 1)),
        ],
        out_specs=[
            pl.BlockSpec((_N, _BB), lambda i: (0, i)),
            pl.BlockSpec((1, 1), lambda i: (0, 0)),
        ],
        out_shape=[
            jax.ShapeDtypeStruct((_N, B), jnp.float32),
            jax.ShapeDtypeStruct((1, 1), jnp.float32),
        ],
        scratch_shapes=[pltpu.VMEM((_N, _N), jnp.float32)],
    )(state_t, src, dst, pos,
      w1a, w1b, W2, W3,
      b1.reshape(1, 128), b2.reshape(1, 128), b3.reshape(1, 128),
      l1g, l1s, l1p, l1t, lb1.reshape(1, 256),
      lw2, lb2.reshape(1, 256), lw3.reshape(1, 256),
      lb3.reshape(1, 1))
    return action.T, regsum[0, 0]
